# Initial kernel scaffold; baseline (speedup 1.0000x reference)
import jax
import jax.numpy as jnp
from jax.experimental import pallas as pl


def kernel(x, pe, gamma, beta, W, b):
    raise NotImplementedError("write your pallas kernel here")


# speedup vs baseline: 4.2612x; 4.2612x over previous
import functools

import jax
import jax.numpy as jnp
from jax import lax
from jax.experimental import pallas as pl
from jax.experimental.pallas import tpu as pltpu
from jax.experimental.pallas import tpu_sc as plsc

EMB_DIM = 64
EMB_NUM = 12
NPAIR = EMB_NUM * EMB_NUM
PD = 2 * EMB_DIM

NC = 2
NS = 16
NW = NC * NS

IDXW = 128
K = 4
CHUNK = K * IDXW


def _prep_body(pe_ref, gamma_ref, beta_ref, w_ref, b_ref, xa_ref, xb_ref,
               tab_ref, pidx_ref):
    n = EMB_NUM
    row = lax.broadcasted_iota(jnp.int32, (n, n), 0)
    col = lax.broadcasted_iota(jnp.int32, (n, n), 1)
    h = jnp.where(row == col, jnp.float32(1.0), jnp.float32(0.0)) + jnp.float32(0.05)
    mean = jnp.mean(h, axis=1, keepdims=True)
    var = jnp.mean((h - mean) ** 2, axis=1, keepdims=True)
    hn = (h - mean) / jnp.sqrt(var + jnp.float32(1e-5))
    hn = hn * gamma_ref[...] + beta_ref[...]
    t = lax.dot_general(hn, w_ref[...], (((1,), (1,)), ((), ())),
                        preferred_element_type=jnp.float32)
    t = jnp.maximum(t + b_ref[...], jnp.float32(0.0)) + pe_ref[...]

    p_iota = lax.broadcasted_iota(jnp.int32, (NPAIR, n), 0)
    c_iota = lax.broadcasted_iota(jnp.int32, (NPAIR, n), 1)
    sel_a = (p_iota // n == c_iota).astype(jnp.float32)
    sel_b = (p_iota % n == c_iota).astype(jnp.float32)
    tab_ref[:, :EMB_DIM] = lax.dot_general(
        sel_a, t, (((1,), (0,)), ((), ())), preferred_element_type=jnp.float32)
    tab_ref[:, EMB_DIM:] = lax.dot_general(
        sel_b, t, (((1,), (0,)), ((), ())), preferred_element_type=jnp.float32)

    pidx_ref[...] = xa_ref[...] * n + xb_ref[...]


def _prep(pe, gamma, beta, W, b, xa, xb):
    return pl.pallas_call(
        _prep_body,
        out_shape=[
            jax.ShapeDtypeStruct((NPAIR, PD), jnp.float32),
            jax.ShapeDtypeStruct(xa.shape, jnp.int32),
        ],
    )(pe, gamma.reshape(1, EMB_NUM), beta.reshape(1, EMB_NUM),
      W, b.reshape(1, EMB_DIM), xa, xb)


def _make_gather(total_pairs):
    assert total_pairs % (NW * CHUNK) == 0
    per_w = total_pairs // NW
    n_chunks = per_w // CHUNK
    mesh = plsc.VectorSubcoreMesh(core_axis_name="c", subcore_axis_name="s")

    @functools.partial(
        pl.kernel,
        mesh=mesh,
        out_type=jax.ShapeDtypeStruct((total_pairs, PD), jnp.float32),
        scratch_types=[
            pltpu.VMEM((CHUNK,), jnp.int32),
            pltpu.VMEM((CHUNK, PD), jnp.float32),
            pltpu.SemaphoreType.DMA,
        ],
    )
    def gather_kernel(table_hbm, idx_hbm, out_hbm, idx_v, rows_v, sem):
        wid = lax.axis_index("s") * NC + lax.axis_index("c")
        base = wid * per_w

        def body(i, carry):
            p0 = base + i * CHUNK
            pltpu.sync_copy(idx_hbm.at[pl.ds(p0, CHUNK)], idx_v)
            copies = []
            for j in range(K):
                copies.append(pltpu.async_copy(
                    table_hbm.at[idx_v.at[pl.ds(j * IDXW, IDXW)]],
                    rows_v.at[pl.ds(j * IDXW, IDXW)],
                    sem))
            for c in copies:
                c.wait()
            pltpu.sync_copy(rows_v, out_hbm.at[pl.ds(p0, CHUNK)])
            return carry

        lax.fori_loop(0, n_chunks, body, 0)

    return gather_kernel


def kernel(x, pe, gamma, beta, W, b):
    Bb, Ll = x.shape
    total_pairs = (Bb * Ll) // 2
    xp = x.reshape(total_pairs, 2).astype(jnp.int32)
    xa = xp[:, 0].reshape(total_pairs // IDXW, IDXW)
    xb = xp[:, 1].reshape(total_pairs // IDXW, IDXW)
    table2, pidx = _prep(pe, gamma, beta, W, b, xa, xb)
    out = _make_gather(total_pairs)(table2, pidx.reshape(total_pairs))
    return out.reshape(Bb, Ll, EMB_DIM)


# speedup vs baseline: 6.6798x; 1.5676x over previous
import functools

import jax
import jax.numpy as jnp
from jax import lax
from jax.experimental import pallas as pl
from jax.experimental.pallas import tpu as pltpu
from jax.experimental.pallas import tpu_sc as plsc

EMB_DIM = 64
EMB_NUM = 12
NPAIR = EMB_NUM * EMB_NUM
PD = 2 * EMB_DIM

NC = 2
NS = 16
NW = NC * NS

IDXW = 128
K = 2
CHUNK = K * IDXW
NBUF = 2


def _prep_body(pe_ref, gamma_ref, beta_ref, w_ref, b_ref, xa_ref, xb_ref,
               tab_ref, pidx_ref):
    n = EMB_NUM
    row = lax.broadcasted_iota(jnp.int32, (n, n), 0)
    col = lax.broadcasted_iota(jnp.int32, (n, n), 1)
    h = jnp.where(row == col, jnp.float32(1.0), jnp.float32(0.0)) + jnp.float32(0.05)
    mean = jnp.mean(h, axis=1, keepdims=True)
    var = jnp.mean((h - mean) ** 2, axis=1, keepdims=True)
    hn = (h - mean) / jnp.sqrt(var + jnp.float32(1e-5))
    hn = hn * gamma_ref[...] + beta_ref[...]
    t = lax.dot_general(hn, w_ref[...], (((1,), (1,)), ((), ())),
                        preferred_element_type=jnp.float32)
    t = jnp.maximum(t + b_ref[...], jnp.float32(0.0)) + pe_ref[...]

    p_iota = lax.broadcasted_iota(jnp.int32, (NPAIR, n), 0)
    c_iota = lax.broadcasted_iota(jnp.int32, (NPAIR, n), 1)
    sel_a = (p_iota // n == c_iota).astype(jnp.float32)
    sel_b = (p_iota % n == c_iota).astype(jnp.float32)
    tab_ref[:, :EMB_DIM] = lax.dot_general(
        sel_a, t, (((1,), (0,)), ((), ())), preferred_element_type=jnp.float32)
    tab_ref[:, EMB_DIM:] = lax.dot_general(
        sel_b, t, (((1,), (0,)), ((), ())), preferred_element_type=jnp.float32)

    pidx_ref[...] = xa_ref[...] * n + xb_ref[...]


def _prep(pe, gamma, beta, W, b, xa, xb):
    return pl.pallas_call(
        _prep_body,
        out_shape=[
            jax.ShapeDtypeStruct((NPAIR, PD), jnp.float32),
            jax.ShapeDtypeStruct(xa.shape, jnp.int32),
        ],
    )(pe, gamma.reshape(1, EMB_NUM), beta.reshape(1, EMB_NUM),
      W, b.reshape(1, EMB_DIM), xa, xb)


def _make_gather(total_pairs):
    assert total_pairs % (NW * CHUNK * NBUF) == 0
    per_w = total_pairs // NW
    n_groups = per_w // (CHUNK * NBUF)
    mesh = plsc.VectorSubcoreMesh(core_axis_name="c", subcore_axis_name="s")

    @functools.partial(
        pl.kernel,
        mesh=mesh,
        out_type=jax.ShapeDtypeStruct((total_pairs, PD), jnp.float32),
        scratch_types=[
            pltpu.VMEM_SHARED((NPAIR, PD), jnp.float32),
            pltpu.VMEM((per_w,), jnp.int32),
            [pltpu.VMEM((CHUNK, PD), jnp.float32) for _ in range(NBUF)],
            pltpu.SemaphoreType.DMA,
            [pltpu.SemaphoreType.DMA for _ in range(NBUF)],
        ],
    )
    def gather_kernel(table_hbm, idx_hbm, out_hbm, table_v, idx_v, rows,
                      gsem, ssems):
        sid = lax.axis_index("s")
        wid = sid * NC + lax.axis_index("c")
        base = wid * per_w
        @pl.when(sid == 0)
        def _stage_table():
            pltpu.sync_copy(table_hbm, table_v)

        pltpu.sync_copy(idx_hbm.at[pl.ds(base, per_w)], idx_v)
        plsc.subcore_barrier()

        def group(g, carry):
            for bf in range(NBUF):
                off = (g * NBUF + bf) * CHUNK

                @pl.when(g > 0)
                def _wait_prev_store():
                    pltpu.make_async_copy(
                        rows[bf], out_hbm.at[pl.ds(base + off, CHUNK)],
                        ssems[bf]).wait()

                copies = []
                for j in range(K):
                    copies.append(pltpu.async_copy(
                        table_v.at[idx_v.at[pl.ds(off + j * IDXW, IDXW)]],
                        rows[bf].at[pl.ds(j * IDXW, IDXW)],
                        gsem))
                for c in copies:
                    c.wait()
                pltpu.async_copy(rows[bf], out_hbm.at[pl.ds(base + off, CHUNK)],
                                 ssems[bf])
            return carry

        lax.fori_loop(0, n_groups, group, 0)
        for bf in range(NBUF):
            pltpu.make_async_copy(
                rows[bf], out_hbm.at[pl.ds(base, CHUNK)], ssems[bf]).wait()

    return gather_kernel


def kernel(x, pe, gamma, beta, W, b):
    Bb, Ll = x.shape
    total_pairs = (Bb * Ll) // 2
    xp = x.reshape(total_pairs, 2).astype(jnp.int32)
    xa = xp[:, 0].reshape(total_pairs // IDXW, IDXW)
    xb = xp[:, 1].reshape(total_pairs // IDXW, IDXW)
    table2, pidx = _prep(pe, gamma, beta, W, b, xa, xb)
    out = _make_gather(total_pairs)(table2, pidx.reshape(total_pairs))
    return out.reshape(Bb, Ll, EMB_DIM)


# speedup vs baseline: 6.7077x; 1.0042x over previous
import functools

import jax
import jax.numpy as jnp
from jax import lax
from jax.experimental import pallas as pl
from jax.experimental.pallas import tpu as pltpu
from jax.experimental.pallas import tpu_sc as plsc

EMB_DIM = 64
EMB_NUM = 12
NPAIR = EMB_NUM * EMB_NUM
PD = 2 * EMB_DIM

NC = 2
NS = 16
NW = NC * NS

IDXW = 128
CHUNK = IDXW
NBUF = 5


def _prep_body(pe_ref, gamma_ref, beta_ref, w_ref, b_ref, xa_ref, xb_ref,
               tab_ref, pidx_ref):
    n = EMB_NUM
    row = lax.broadcasted_iota(jnp.int32, (n, n), 0)
    col = lax.broadcasted_iota(jnp.int32, (n, n), 1)
    h = jnp.where(row == col, jnp.float32(1.0), jnp.float32(0.0)) + jnp.float32(0.05)
    mean = jnp.mean(h, axis=1, keepdims=True)
    var = jnp.mean((h - mean) ** 2, axis=1, keepdims=True)
    hn = (h - mean) / jnp.sqrt(var + jnp.float32(1e-5))
    hn = hn * gamma_ref[...] + beta_ref[...]
    t = lax.dot_general(hn, w_ref[...], (((1,), (1,)), ((), ())),
                        preferred_element_type=jnp.float32)
    t = jnp.maximum(t + b_ref[...], jnp.float32(0.0)) + pe_ref[...]

    p_iota = lax.broadcasted_iota(jnp.int32, (NPAIR, n), 0)
    c_iota = lax.broadcasted_iota(jnp.int32, (NPAIR, n), 1)
    sel_a = (p_iota // n == c_iota).astype(jnp.float32)
    sel_b = (p_iota % n == c_iota).astype(jnp.float32)
    tab_ref[:, :EMB_DIM] = lax.dot_general(
        sel_a, t, (((1,), (0,)), ((), ())), preferred_element_type=jnp.float32)
    tab_ref[:, EMB_DIM:] = lax.dot_general(
        sel_b, t, (((1,), (0,)), ((), ())), preferred_element_type=jnp.float32)

    pidx_ref[...] = xa_ref[...] * n + xb_ref[...]


def _prep(pe, gamma, beta, W, b, xa, xb):
    return pl.pallas_call(
        _prep_body,
        out_shape=[
            jax.ShapeDtypeStruct((NPAIR, PD), jnp.float32),
            jax.ShapeDtypeStruct(xa.shape, jnp.int32),
        ],
    )(pe, gamma.reshape(1, EMB_NUM), beta.reshape(1, EMB_NUM),
      W, b.reshape(1, EMB_DIM), xa, xb)


def _make_gather(total_pairs):
    assert total_pairs % (NW * CHUNK * NBUF) == 0
    per_w = total_pairs // NW
    n_groups = per_w // (CHUNK * NBUF)
    mesh = plsc.VectorSubcoreMesh(core_axis_name="c", subcore_axis_name="s")

    @functools.partial(
        pl.kernel,
        mesh=mesh,
        out_type=jax.ShapeDtypeStruct((total_pairs, PD), jnp.float32),
        scratch_types=[
            pltpu.VMEM_SHARED((NPAIR, PD), jnp.float32),
            pltpu.VMEM((per_w,), jnp.int32),
            [pltpu.VMEM((CHUNK, PD), jnp.float32) for _ in range(NBUF)],
            [pltpu.SemaphoreType.DMA for _ in range(NBUF)],
            [pltpu.SemaphoreType.DMA for _ in range(NBUF)],
        ],
    )
    def gather_kernel(table_hbm, idx_hbm, out_hbm, table_v, idx_v, rows,
                      gsems, ssems):
        sid = lax.axis_index("s")
        wid = sid * NC + lax.axis_index("c")
        base = wid * per_w
        @pl.when(sid == 0)
        def _stage_table():
            pltpu.sync_copy(table_hbm, table_v)

        pltpu.sync_copy(idx_hbm.at[pl.ds(base, per_w)], idx_v)
        plsc.subcore_barrier()

        def group(g, carry):
            for bf in range(NBUF):
                off = (g * NBUF + bf) * CHUNK

                @pl.when(g > 0)
                def _wait_prev_store():
                    pltpu.make_async_copy(
                        rows[bf], out_hbm.at[pl.ds(base + off, CHUNK)],
                        ssems[bf]).wait()

                pltpu.async_copy(
                    table_v.at[idx_v.at[pl.ds(off, CHUNK)]],
                    rows[bf], gsems[bf])
            for bf in range(NBUF):
                off = (g * NBUF + bf) * CHUNK
                pltpu.make_async_copy(
                    table_v.at[idx_v.at[pl.ds(off, CHUNK)]],
                    rows[bf], gsems[bf]).wait()
                pltpu.async_copy(rows[bf], out_hbm.at[pl.ds(base + off, CHUNK)],
                                 ssems[bf])
            return carry

        lax.fori_loop(0, n_groups, group, 0)
        for bf in range(NBUF):
            pltpu.make_async_copy(
                rows[bf], out_hbm.at[pl.ds(base, CHUNK)], ssems[bf]).wait()

    return gather_kernel


def kernel(x, pe, gamma, beta, W, b):
    Bb, Ll = x.shape
    total_pairs = (Bb * Ll) // 2
    xp = x.reshape(total_pairs, 2).astype(jnp.int32)
    xa = xp[:, 0].reshape(total_pairs // IDXW, IDXW)
    xb = xp[:, 1].reshape(total_pairs // IDXW, IDXW)
    table2, pidx = _prep(pe, gamma, beta, W, b, xa, xb)
    out = _make_gather(total_pairs)(table2, pidx.reshape(total_pairs))
    return out.reshape(Bb, Ll, EMB_DIM)
